# baseline (speedup 1.0000x reference)
import jax
import jax.numpy as jnp
from jax.experimental import pallas as pl

N = 4096
G = 4
D = 256
THR = -1.0
BM = 256
BN = 4096


def _h_kernel(x_ref, w_ref, b_ref, h_ref, sq_ref):
    x = x_ref[...]
    h = jnp.dot(x, w_ref[...], preferred_element_type=jnp.float32) + b_ref[...]
    h_ref[...] = h
    hh = h * h
    ones = jnp.ones((1, D), jnp.float32)
    sq_ref[...] = jax.lax.dot_general(
        ones, hh, (((1,), (1,)), ((), ())), preferred_element_type=jnp.float32
    )


def _score_kernel(hi_ref, hj_ref, sq_ref, seg_ref, score_ref, adj_ref, bne_ref):
    i = pl.program_id(0)
    j = pl.program_id(1)
    hi = hi_ref[...]
    hj = hj_ref[...]
    dot = jax.lax.dot_general(
        hi, hj, (((1,), (1,)), ((), ())), preferred_element_type=jnp.float32
    )
    sqi = jnp.sum(hi * hi, axis=1, keepdims=True)
    sqj = sq_ref[...]
    score = 2.0 * dot - sqi - sqj
    score_ref[...] = score

    seg_full = seg_ref[...]
    ends = []
    e = jnp.int32(0)
    for k in range(G):
        e = e + jnp.sum((seg_full == k).astype(jnp.int32))
        ends.append(e)
    row = i * BM + jax.lax.broadcasted_iota(jnp.int32, (BM, 1), 0)
    col = j * BN + jax.lax.broadcasted_iota(jnp.int32, (1, BN), 1)
    segr = sum((row >= ends[k]).astype(jnp.int32) for k in range(G))
    segc = sum((col >= ends[k]).astype(jnp.int32) for k in range(G))
    adj = (score > THR) & (segr == segc) & (row != col)
    adj_ref[...] = adj.astype(jnp.int8)

    rowdeg = jnp.sum(adj.astype(jnp.int32), axis=1, keepdims=True)
    lanes = jax.lax.broadcasted_iota(jnp.int32, (1, 128), 1)
    contrib = jnp.sum(jnp.where(segr == lanes, rowdeg, 0), axis=0, keepdims=True)

    @pl.when((i == 0) & (j == 0))
    def _():
        bne_ref[...] = jnp.zeros((1, 1, 128), jnp.int32)

    bne_ref[...] += contrib.reshape(1, 1, 128)


def kernel(t, dynamicVariable, segment_ids, W, b):
    x = dynamicVariable
    b2 = b.reshape(1, D)
    seg2d = segment_ids.reshape(1, N).astype(jnp.int32)

    h, sq = pl.pallas_call(
        _h_kernel,
        grid=(N // BM,),
        in_specs=[
            pl.BlockSpec((BM, D), lambda i: (i, 0)),
            pl.BlockSpec((D, D), lambda i: (0, 0)),
            pl.BlockSpec((1, D), lambda i: (0, 0)),
        ],
        out_specs=[
            pl.BlockSpec((BM, D), lambda i: (i, 0)),
            pl.BlockSpec((1, BM), lambda i: (0, i)),
        ],
        out_shape=[
            jax.ShapeDtypeStruct((N, D), jnp.float32),
            jax.ShapeDtypeStruct((1, N), jnp.float32),
        ],
    )(x, W, b2)

    score, adj, bne3 = pl.pallas_call(
        _score_kernel,
        grid=(N // BM, N // BN),
        in_specs=[
            pl.BlockSpec((BM, D), lambda i, j: (i, 0)),
            pl.BlockSpec((BN, D), lambda i, j: (j, 0)),
            pl.BlockSpec((1, BN), lambda i, j: (0, j)),
            pl.BlockSpec((1, N), lambda i, j: (0, 0)),
        ],
        out_specs=[
            pl.BlockSpec((BM, BN), lambda i, j: (i, j)),
            pl.BlockSpec((BM, BN), lambda i, j: (i, j)),
            pl.BlockSpec((1, 1, 128), lambda i, j: (0, 0, 0)),
        ],
        out_shape=[
            jax.ShapeDtypeStruct((N, N), jnp.float32),
            jax.ShapeDtypeStruct((N, N), jnp.int8),
            jax.ShapeDtypeStruct((1, 1, 128), jnp.int32),
        ],
    )(h, h, sq, seg2d)

    bne = bne3.reshape(128)[:G]
    return (score, adj.astype(jnp.bool_), bne)


# speedup vs baseline: 1.0800x; 1.0800x over previous
import jax
import jax.numpy as jnp
from jax.experimental import pallas as pl

N = 4096
G = 4
D = 256
THR = -1.0
BM = 512
BN = 4096


def _h_kernel(x_ref, w_ref, b_ref, h_ref, sq_ref):
    x = x_ref[...]
    h = jnp.dot(x, w_ref[...], preferred_element_type=jnp.float32) + b_ref[...]
    h_ref[...] = h
    hh = h * h
    ones = jnp.ones((1, D), jnp.float32)
    sq_ref[...] = jax.lax.dot_general(
        ones, hh, (((1,), (1,)), ((), ())), preferred_element_type=jnp.float32
    )


def _score_kernel(hi_ref, hj_ref, sq_ref, seg_ref, score_ref, adj_ref, bne_ref):
    i = pl.program_id(0)
    j = pl.program_id(1)
    hi = hi_ref[...]
    hj = hj_ref[...]
    dot = jax.lax.dot_general(
        hi, hj, (((1,), (1,)), ((), ())), preferred_element_type=jnp.float32
    )
    sqi = jnp.sum(hi * hi, axis=1, keepdims=True)
    sqj = sq_ref[...]
    score = 2.0 * dot - sqi - sqj
    score_ref[...] = score

    seg_full = seg_ref[...]
    ends = []
    e = jnp.int32(0)
    for k in range(G):
        e = e + jnp.sum((seg_full == k).astype(jnp.int32))
        ends.append(e)
    row = i * BM + jax.lax.broadcasted_iota(jnp.int32, (BM, 1), 0)
    col = j * BN + jax.lax.broadcasted_iota(jnp.int32, (1, BN), 1)
    segr = sum((row >= ends[k]).astype(jnp.int32) for k in range(G))
    segc = sum((col >= ends[k]).astype(jnp.int32) for k in range(G))
    adj = (score > THR) & (segr == segc) & (row != col)
    adj_ref[...] = adj.astype(jnp.int8)

    rowdeg = jnp.sum(adj.astype(jnp.int32), axis=1, keepdims=True)
    lanes = jax.lax.broadcasted_iota(jnp.int32, (1, 128), 1)
    contrib = jnp.sum(jnp.where(segr == lanes, rowdeg, 0), axis=0, keepdims=True)

    @pl.when((i == 0) & (j == 0))
    def _():
        bne_ref[...] = jnp.zeros((1, 1, 128), jnp.int32)

    bne_ref[...] += contrib.reshape(1, 1, 128)


def kernel(t, dynamicVariable, segment_ids, W, b):
    x = dynamicVariable
    b2 = b.reshape(1, D)
    seg2d = segment_ids.reshape(1, N).astype(jnp.int32)

    h, sq = pl.pallas_call(
        _h_kernel,
        grid=(N // BM,),
        in_specs=[
            pl.BlockSpec((BM, D), lambda i: (i, 0)),
            pl.BlockSpec((D, D), lambda i: (0, 0)),
            pl.BlockSpec((1, D), lambda i: (0, 0)),
        ],
        out_specs=[
            pl.BlockSpec((BM, D), lambda i: (i, 0)),
            pl.BlockSpec((1, BM), lambda i: (0, i)),
        ],
        out_shape=[
            jax.ShapeDtypeStruct((N, D), jnp.float32),
            jax.ShapeDtypeStruct((1, N), jnp.float32),
        ],
    )(x, W, b2)

    score, adj, bne3 = pl.pallas_call(
        _score_kernel,
        grid=(N // BM, N // BN),
        in_specs=[
            pl.BlockSpec((BM, D), lambda i, j: (i, 0)),
            pl.BlockSpec((BN, D), lambda i, j: (j, 0)),
            pl.BlockSpec((1, BN), lambda i, j: (0, j)),
            pl.BlockSpec((1, N), lambda i, j: (0, 0)),
        ],
        out_specs=[
            pl.BlockSpec((BM, BN), lambda i, j: (i, j)),
            pl.BlockSpec((BM, BN), lambda i, j: (i, j)),
            pl.BlockSpec((1, 1, 128), lambda i, j: (0, 0, 0)),
        ],
        out_shape=[
            jax.ShapeDtypeStruct((N, N), jnp.float32),
            jax.ShapeDtypeStruct((N, N), jnp.int8),
            jax.ShapeDtypeStruct((1, 1, 128), jnp.int32),
        ],
    )(h, h, sq, seg2d)

    bne = bne3.reshape(128)[:G]
    return (score, adj.astype(jnp.bool_), bne)


# speedup vs baseline: 1.2122x; 1.1224x over previous
import jax
import jax.numpy as jnp
from jax.experimental import pallas as pl
from jax.experimental.pallas import tpu as pltpu

N = 4096
G = 4
D = 256
THR = -1.0
BM = 512


def _edge_kernel(
    x_ref, w_ref, b_ref, seg_ref, score_ref, adj_ref, bne_ref, h_scr, sq_scr, ends_scr
):
    t = pl.program_id(0)

    @pl.when(t == 0)
    def _():
        x = x_ref[...]
        h = jnp.dot(x, w_ref[...], preferred_element_type=jnp.float32) + b_ref[...]
        h_scr[...] = h
        ones = jnp.ones((1, D), jnp.float32)
        sq_scr[...] = jax.lax.dot_general(
            ones, h * h, (((1,), (1,)), ((), ())), preferred_element_type=jnp.float32
        )
        seg_full = seg_ref[...]
        e = jnp.int32(0)
        for k in range(G):
            e = e + jnp.sum((seg_full == k).astype(jnp.int32))
            ends_scr[k] = e
        bne_ref[...] = jnp.zeros((1, 1, 128), jnp.int32)

    @pl.when(t > 0)
    def _():
        i = t - 1
        hi = h_scr[pl.ds(i * BM, BM), :]
        hfull = h_scr[...]
        dot = jax.lax.dot_general(
            hi + hi, hfull, (((1,), (1,)), ((), ())), preferred_element_type=jnp.float32
        )
        sqi = jnp.sum(hi * hi, axis=1, keepdims=True)
        score = dot - sqi - sq_scr[...]
        score_ref[...] = score

        ends = [ends_scr[k] for k in range(G)]
        row = i * BM + jax.lax.broadcasted_iota(jnp.int32, (BM, 1), 0)
        col = jax.lax.broadcasted_iota(jnp.int32, (1, N), 1)
        segr = sum((row >= ends[k]).astype(jnp.int32) for k in range(G))
        segc = sum((col >= ends[k]).astype(jnp.int32) for k in range(G))
        adj = (score > THR) & (segr == segc) & (row != col)
        adj_ref[...] = adj.astype(jnp.int8)

        rowdeg = jnp.sum(adj.astype(jnp.int32), axis=1, keepdims=True)
        lanes = jax.lax.broadcasted_iota(jnp.int32, (1, 128), 1)
        contrib = jnp.sum(jnp.where(segr == lanes, rowdeg, 0), axis=0, keepdims=True)
        bne_ref[...] += contrib.reshape(1, 1, 128)


def kernel(t, dynamicVariable, segment_ids, W, b):
    x = dynamicVariable
    b2 = b.reshape(1, D)
    seg2d = segment_ids.reshape(1, N).astype(jnp.int32)

    nb = N // BM
    score, adj, bne3 = pl.pallas_call(
        _edge_kernel,
        grid=(nb + 1,),
        in_specs=[
            pl.BlockSpec((N, D), lambda t: (0, 0)),
            pl.BlockSpec((D, D), lambda t: (0, 0)),
            pl.BlockSpec((1, D), lambda t: (0, 0)),
            pl.BlockSpec((1, N), lambda t: (0, 0)),
        ],
        out_specs=[
            pl.BlockSpec((BM, N), lambda t: (jnp.maximum(t - 1, 0), 0)),
            pl.BlockSpec((BM, N), lambda t: (jnp.maximum(t - 1, 0), 0)),
            pl.BlockSpec((1, 1, 128), lambda t: (0, 0, 0)),
        ],
        out_shape=[
            jax.ShapeDtypeStruct((N, N), jnp.float32),
            jax.ShapeDtypeStruct((N, N), jnp.int8),
            jax.ShapeDtypeStruct((1, 1, 128), jnp.int32),
        ],
        scratch_shapes=[
            pltpu.VMEM((N, D), jnp.float32),
            pltpu.VMEM((1, N), jnp.float32),
            pltpu.SMEM((G,), jnp.int32),
        ],
    )(x, W, b2, seg2d)

    bne = bne3.reshape(128)[:G]
    return (score, adj.astype(jnp.bool_), bne)
